# Initial kernel scaffold; baseline (speedup 1.0000x reference)
import jax
import jax.numpy as jnp
from jax.experimental import pallas as pl


def kernel(src_v, dst_u, x_v, x_u, shuff_x_v, shuff_x_u, W1v, W1u, b1v, b1u, alpha1, W2v, W2u, b2v, b2u, Wb):
    raise NotImplementedError("write your pallas kernel here")


# speedup vs baseline: 8.6462x; 8.6462x over previous
import functools

import jax
import jax.numpy as jnp
from jax import lax
from jax.experimental import pallas as pl
from jax.experimental.pallas import tpu as pltpu
from jax.experimental.pallas import tpu_sc as plsc

_NV, _NU, _E = 10000, 2000, 320000
_D = 128
_CH = 40
_NBLK = 25
_BLK = 20
_NBUF = 4
_TILES = 16
_CH2 = 80
_CPT2 = _E // _CH2 // 32
_NBUF2 = 5


def _mm_pair_body(xv_ref, xu_ref, wv_ref, wu_ref, ov_ref, ou_ref):
    ov_ref[...] = jnp.dot(xv_ref[...], wv_ref[...],
                          preferred_element_type=jnp.float32)
    ou_ref[...] = jnp.dot(xu_ref[...], wu_ref[...],
                          preferred_element_type=jnp.float32)


def _tc_mm_pair(x_v, x_u, wv, wu):
    return pl.pallas_call(
        _mm_pair_body,
        out_shape=(jax.ShapeDtypeStruct((_NV, 2 * _D), jnp.float32),
                   jax.ShapeDtypeStruct((_NU, 2 * _D), jnp.float32)),
    )(x_v, x_u, wv, wu)


def _layer2_body(sv_ref, su_ref, gxv_ref, gxu_ref, wv_ref, wu_ref,
                 bv_ref, bu_ref, al_ref, gv_ref, gu_ref):
    al = al_ref[...]
    av = sv_ref[0:_NV, :] + bv_ref[...]
    av = jnp.where(av >= 0.0, av, al * av)
    gv_ref[...] = jnp.dot(av, wv_ref[...],
                          preferred_element_type=jnp.float32) + gxv_ref[...]
    au = su_ref[0:_NU, :] + bu_ref[...]
    au = jnp.where(au >= 0.0, au, al * au)
    gu_ref[...] = jnp.dot(au, wu_ref[...],
                          preferred_element_type=jnp.float32) + gxu_ref[...]


def _tc_layer2(sv1, su1, gxv, gxu, w2v_top, w2u_top, b1v, b1u, alpha):
    return pl.pallas_call(
        _layer2_body,
        out_shape=(jax.ShapeDtypeStruct((_NV, _D), jnp.float32),
                   jax.ShapeDtypeStruct((_NU, _D), jnp.float32)),
    )(sv1, su1, gxv, gxu, w2v_top, w2u_top, b1v, b1u, alpha)


def _disc_prep_body(sv_ref, su_ref, bv_ref, bu_ref, wb_ref,
                    ve2_ref, ue2_ref, av_ref):
    v = sv_ref[0:_NV, :] + bv_ref[...]
    ve2_ref[...] = v
    ue2_ref[...] = su_ref[0:_NU, :] + bu_ref[...]
    av_ref[...] = jnp.dot(v, wb_ref[...], preferred_element_type=jnp.float32)


def _tc_disc_prep(sv2, su2, b2v, b2u, wb):
    return pl.pallas_call(
        _disc_prep_body,
        out_shape=(jax.ShapeDtypeStruct((_NV, _D), jnp.float32),
                   jax.ShapeDtypeStruct((_NU, _D), jnp.float32),
                   jax.ShapeDtypeStruct((_NV, _D), jnp.float32)),
    )(sv2, su2, b2v, b2u, wb)


def _ls(x):
    return jnp.minimum(x, 0.0) - jnp.log(1.0 + jnp.exp(-jnp.abs(x)))


_BM = 400


def _logits_body(av_ref, u_ref, l_ref, s0_ref):
    band = lax.dot_general(av_ref[...], u_ref[...],
                           (((1,), (1,)), ((), ())),
                           preferred_element_type=jnp.float32)
    l_ref[...] = band
    part = jnp.sum(_ls(-band))
    pid = pl.program_id(0)
    s0_ref[...] = jnp.where(pid == 0, part, s0_ref[...] + part)


def _tc_logits(av, ue2):
    nb = _NV // _BM
    return pl.pallas_call(
        _logits_body,
        grid=(nb,),
        in_specs=[pl.BlockSpec((_BM, _D), lambda i: (i, 0)),
                  pl.BlockSpec((_NU, _D), lambda i: (0, 0))],
        out_specs=(pl.BlockSpec((_BM, _NU), lambda i: (i, 0)),
                   pl.BlockSpec((1, 1), lambda i: (0, 0))),
        out_shape=(jax.ShapeDtypeStruct((_NV, _NU), jnp.float32),
                   jax.ShapeDtypeStruct((1, 1), jnp.float32)),
    )(av, ue2)


def _edge_reduce_body(l_ref, sneg_ref, spos_ref):
    x = l_ref[...]
    sneg_ref[...] = jnp.sum(_ls(-x)).reshape(1, 1)
    spos_ref[...] = jnp.sum(_ls(x)).reshape(1, 1)


def _tc_edge_reduce(l2d):
    return pl.pallas_call(
        _edge_reduce_body,
        out_shape=(jax.ShapeDtypeStruct((1, 1), jnp.float32),
                   jax.ShapeDtypeStruct((1, 1), jnp.float32)),
    )(l2d)


def _sc_segsum_pair(src4d, dst4d, table_u, table_v):
    mesh = plsc.VectorSubcoreMesh(core_axis_name="c", subcore_axis_name="s")

    @functools.partial(
        pl.kernel, mesh=mesh,
        out_type=(jax.ShapeDtypeStruct((_NV, _D), jnp.float32),
                  jax.ShapeDtypeStruct((_NU, _D), jnp.float32)),
        scratch_types=[
            pltpu.VMEM((_BLK, _CH), jnp.int32),
            pltpu.VMEM((_BLK, _CH), jnp.int32),
            pltpu.VMEM((_CH, _D), jnp.float32),
            pltpu.VMEM((_CH, _D), jnp.float32),
            pltpu.VMEM((_CH, _D), jnp.float32),
            pltpu.VMEM((_CH, _D), jnp.float32),
            pltpu.SemaphoreType.DMA,
            pltpu.SemaphoreType.DMA,
            pltpu.SemaphoreType.DMA,
            pltpu.SemaphoreType.DMA,
            pltpu.VMEM_SHARED((_NV, _D), jnp.float32),
        ],
    )
    def k(src_hbm, dst_hbm, tu_hbm, tv_hbm, outv_hbm, outu_hbm,
          gidx, sidx, r0, r1, r2, r3, g0, g1, g2, g3, acc):
        rows = (r0, r1, r2, r3)
        gsem = (g0, g1, g2, g3)
        c = lax.axis_index("c")
        s = lax.axis_index("s")

        def _zrow(r, carry):
            for j in range(_D // 16):
                r0[r, pl.ds(j * 16, 16)] = jnp.zeros((16,), jnp.float32)
            return carry
        lax.fori_loop(0, _CH, _zrow, 0)

        def run_dir(g4d_hbm, s4d_hbm, table_hbm, out_hbm,
                    per_tile, plan, extra_plan):
            base = s * per_tile
            off = 0
            for ch in plan:
                pltpu.sync_copy(r0.at[pl.ds(0, ch)],
                                acc.at[pl.ds(base + off, ch)])
                off += ch

            @pl.when(s == 15)
            def _():
                o = per_tile
                for ch in extra_plan:
                    pltpu.sync_copy(r0.at[pl.ds(0, ch)],
                                    acc.at[pl.ds(15 * per_tile + o, ch)])
                    o += ch
            plsc.subcore_barrier()

            g4d_t = g4d_hbm.at[s]
            s4d_t = s4d_hbm.at[s]

            def block(blk, carry):
                pltpu.sync_copy(g4d_t.at[blk], gidx)
                pltpu.sync_copy(s4d_t.at[blk], sidx)
                for b in range(_NBUF):
                    pltpu.async_copy(table_hbm.at[gidx.at[b]],
                                     rows[b], gsem[b])

                def inner(io, cy):
                    ibase = io * _NBUF
                    for b in range(_NBUF):
                        i = ibase + b
                        pltpu.make_async_copy(
                            table_hbm.at[pl.ds(0, _CH)],
                            rows[b], gsem[b]).wait()
                        pltpu.sync_copy(rows[b], acc.at[sidx.at[i]],
                                        add=True)
                        nxt = i + _NBUF

                        @pl.when(nxt < _BLK)
                        def _():
                            pltpu.async_copy(table_hbm.at[gidx.at[nxt]],
                                             rows[b], gsem[b])
                    return cy
                lax.fori_loop(0, _BLK // _NBUF, inner, 0)
                return carry
            lax.fori_loop(0, _NBLK, block, 0)
            plsc.subcore_barrier()

            def copy_out(row0, ch):
                pltpu.sync_copy(acc.at[pl.ds(row0, ch)],
                                r0.at[pl.ds(0, ch)])
                pltpu.sync_copy(r0.at[pl.ds(0, ch)],
                                out_hbm.at[pl.ds(row0, ch)])

            off = 0
            for ch in plan:
                copy_out(base + off, ch)
                off += ch

            @pl.when(s == 15)
            def _():
                o = per_tile
                for ch in extra_plan:
                    copy_out(15 * per_tile + o, ch)
                    o += ch

        @pl.when(c == 0)
        def _():
            run_dir(dst_hbm, src_hbm, tu_hbm, outv_hbm,
                    624, (40,) * 15 + (24,), (16,))

        @pl.when(c == 1)
        def _():
            run_dir(src_hbm, dst_hbm, tv_hbm, outu_hbm,
                    120, (40, 40, 40), (40, 40))

    return k(src4d, dst4d, table_u, table_v)


def _sc_edge_gather(lflat, src3d2, dst3d2):
    mesh = plsc.VectorSubcoreMesh(core_axis_name="c", subcore_axis_name="s")

    @functools.partial(
        pl.kernel, mesh=mesh,
        out_type=jax.ShapeDtypeStruct((32, _CPT2, _CH2), jnp.float32),
        scratch_types=[
            pltpu.VMEM((_CPT2, _CH2), jnp.int32),
            pltpu.VMEM((_CPT2, _CH2), jnp.int32),
            pltpu.VMEM((_CPT2, _CH2), jnp.int32),
            pltpu.VMEM((_CPT2, _CH2), jnp.float32),
            pltpu.SemaphoreType.DMA,
            pltpu.SemaphoreType.DMA,
            pltpu.SemaphoreType.DMA,
            pltpu.SemaphoreType.DMA,
            pltpu.SemaphoreType.DMA,
        ],
    )
    def k(lflat_hbm, src_hbm, dst_hbm, out_hbm,
          sbuf, dbuf, kbuf, lbuf, g0, g1, g2, g3, g4):
        gsem = (g0, g1, g2, g3, g4)
        c = lax.axis_index("c")
        s = lax.axis_index("s")
        w = c * _TILES + s
        pltpu.sync_copy(src_hbm.at[w], sbuf)
        pltpu.sync_copy(dst_hbm.at[w], dbuf)

        def krow(r, carry):
            for j in range(_CH2 // 16):
                sl = pl.ds(j * 16, 16)
                kbuf[r, sl] = sbuf[r, sl] * _NU + dbuf[r, sl]
            return carry
        lax.fori_loop(0, _CPT2, krow, 0)

        for b in range(_NBUF2):
            pltpu.async_copy(lflat_hbm.at[kbuf.at[b]], lbuf.at[b], gsem[b])

        def outer(io, carry):
            ibase = io * _NBUF2
            for b in range(_NBUF2):
                i = ibase + b
                pltpu.make_async_copy(
                    lflat_hbm.at[pl.ds(0, _CH2)], lbuf.at[i], gsem[b]).wait()
                nxt = i + _NBUF2

                @pl.when(nxt < _CPT2)
                def _():
                    pltpu.async_copy(
                        lflat_hbm.at[kbuf.at[nxt]], lbuf.at[nxt], gsem[b])
            return carry
        lax.fori_loop(0, _CPT2 // _NBUF2, outer, 0)
        pltpu.sync_copy(lbuf, out_hbm.at[w])

    return k(lflat, src3d2, dst3d2)


def kernel(src_v, dst_u, x_v, x_u, shuff_x_v, shuff_x_u,
           W1v, W1u, b1v, b1u, alpha1, W2v, W2u, b2v, b2u, Wb):
    src4d = src_v.reshape(_TILES, _NBLK, _BLK, _CH)
    dst4d = dst_u.reshape(_TILES, _NBLK, _BLK, _CH)
    src3d2 = src_v.reshape(32, _CPT2, _CH2)
    dst3d2 = dst_u.reshape(32, _CPT2, _CH2)
    b1v_r = b1v.reshape(1, _D)
    b1u_r = b1u.reshape(1, _D)
    b2v_r = b2v.reshape(1, _D)
    b2u_r = b2u.reshape(1, _D)
    al_r = jnp.broadcast_to(alpha1, (1, _D)).astype(jnp.float32)
    wv_cat = jnp.concatenate([W1v, W2v[_D:, :]], axis=1)
    wu_cat = jnp.concatenate([W1u, W2u[_D:, :]], axis=1)

    hv_cat, hu_cat = _tc_mm_pair(x_v, x_u, wv_cat, wu_cat)
    hv, gxv = hv_cat[:, :_D], hv_cat[:, _D:]
    hu, gxu = hu_cat[:, :_D], hu_cat[:, _D:]

    sv1, su1 = _sc_segsum_pair(src4d, dst4d, hu, hv)

    gv, gu = _tc_layer2(sv1, su1, gxv, gxu, W2v[:_D, :], W2u[:_D, :],
                        b1v_r, b1u_r, al_r)

    sv2, su2 = _sc_segsum_pair(src4d, dst4d, gu, gv)

    ve2, ue2, av = _tc_disc_prep(sv2, su2, b2v_r, b2u_r, Wb)

    big_l, s0 = _tc_logits(av, ue2)

    l2d = _sc_edge_gather(big_l.reshape(-1), src3d2, dst3d2)
    sneg, spos = _tc_edge_reduce(l2d.reshape(_E // _D, _D))

    n = float(_NV) * float(_NU)
    tsum = float(_E)
    pos_weight = (n - tsum) / tsum
    norm = n / (n - tsum)
    sum_bce = -s0[0, 0] + sneg[0, 0] - pos_weight * spos[0, 0]
    loss = norm * sum_bce / n
    return ve2, ue2, loss


# speedup vs baseline: 9.5161x; 1.1006x over previous
import functools

import jax
import jax.numpy as jnp
from jax import lax
from jax.experimental import pallas as pl
from jax.experimental.pallas import tpu as pltpu
from jax.experimental.pallas import tpu_sc as plsc

_NV, _NU, _E = 10000, 2000, 320000
_D = 128
_CH = 40
_NBLK = 5
_BLK = 100
_NBUF = 4
_TILES = 16
_CH2 = 80
_CPT2 = _E // _CH2 // 32
_NBUF2 = 5


def _mm_pair_body(xv_ref, xu_ref, wv_ref, wu_ref, ov_ref, ou_ref):
    ov_ref[...] = jnp.dot(xv_ref[...], wv_ref[...],
                          preferred_element_type=jnp.float32)
    ou_ref[...] = jnp.dot(xu_ref[...], wu_ref[...],
                          preferred_element_type=jnp.float32)


def _tc_mm_pair(x_v, x_u, wv, wu):
    return pl.pallas_call(
        _mm_pair_body,
        out_shape=(jax.ShapeDtypeStruct((_NV, 2 * _D), jnp.float32),
                   jax.ShapeDtypeStruct((_NU, 2 * _D), jnp.float32)),
    )(x_v, x_u, wv, wu)


def _layer2_body(sv_ref, su_ref, gxv_ref, gxu_ref, wv_ref, wu_ref,
                 bv_ref, bu_ref, al_ref, gv_ref, gu_ref):
    al = al_ref[...]
    av = sv_ref[0:_NV, :] + bv_ref[...]
    av = jnp.where(av >= 0.0, av, al * av)
    gv_ref[...] = jnp.dot(av, wv_ref[...],
                          preferred_element_type=jnp.float32) + gxv_ref[...]
    au = su_ref[0:_NU, :] + bu_ref[...]
    au = jnp.where(au >= 0.0, au, al * au)
    gu_ref[...] = jnp.dot(au, wu_ref[...],
                          preferred_element_type=jnp.float32) + gxu_ref[...]


def _tc_layer2(sv1, su1, gxv, gxu, w2v_top, w2u_top, b1v, b1u, alpha):
    return pl.pallas_call(
        _layer2_body,
        out_shape=(jax.ShapeDtypeStruct((_NV, _D), jnp.float32),
                   jax.ShapeDtypeStruct((_NU, _D), jnp.float32)),
    )(sv1, su1, gxv, gxu, w2v_top, w2u_top, b1v, b1u, alpha)


def _disc_prep_body(sv_ref, su_ref, bv_ref, bu_ref, wb_ref,
                    ve2_ref, ue2_ref, av_ref):
    v = sv_ref[0:_NV, :] + bv_ref[...]
    ve2_ref[...] = v
    ue2_ref[...] = su_ref[0:_NU, :] + bu_ref[...]
    av_ref[...] = jnp.dot(v, wb_ref[...], preferred_element_type=jnp.float32)


def _tc_disc_prep(sv2, su2, b2v, b2u, wb):
    return pl.pallas_call(
        _disc_prep_body,
        out_shape=(jax.ShapeDtypeStruct((_NV, _D), jnp.float32),
                   jax.ShapeDtypeStruct((_NU, _D), jnp.float32),
                   jax.ShapeDtypeStruct((_NV, _D), jnp.float32)),
    )(sv2, su2, b2v, b2u, wb)


def _ls(x):
    return jnp.minimum(x, 0.0) - jnp.log(1.0 + jnp.exp(-jnp.abs(x)))


_BM = 400


def _logits_body(av_ref, u_ref, l_ref, s0_ref):
    band = lax.dot_general(av_ref[...], u_ref[...],
                           (((1,), (1,)), ((), ())),
                           preferred_element_type=jnp.float32)
    l_ref[...] = band
    part = jnp.sum(_ls(-band))
    pid = pl.program_id(0)
    s0_ref[...] = jnp.where(pid == 0, part, s0_ref[...] + part)


def _tc_logits(av, ue2):
    nb = _NV // _BM
    return pl.pallas_call(
        _logits_body,
        grid=(nb,),
        in_specs=[pl.BlockSpec((_BM, _D), lambda i: (i, 0)),
                  pl.BlockSpec((_NU, _D), lambda i: (0, 0))],
        out_specs=(pl.BlockSpec((_BM, _NU), lambda i: (i, 0)),
                   pl.BlockSpec((1, 1), lambda i: (0, 0))),
        out_shape=(jax.ShapeDtypeStruct((_NV, _NU), jnp.float32),
                   jax.ShapeDtypeStruct((1, 1), jnp.float32)),
    )(av, ue2)


def _edge_reduce_body(l_ref, sneg_ref, spos_ref):
    x = l_ref[...]
    sneg_ref[...] = jnp.sum(_ls(-x)).reshape(1, 1)
    spos_ref[...] = jnp.sum(_ls(x)).reshape(1, 1)


def _tc_edge_reduce(l2d):
    return pl.pallas_call(
        _edge_reduce_body,
        out_shape=(jax.ShapeDtypeStruct((1, 1), jnp.float32),
                   jax.ShapeDtypeStruct((1, 1), jnp.float32)),
    )(l2d)


def _sc_segsum_pair(src4d, dst4d, table_u, table_v):
    mesh = plsc.VectorSubcoreMesh(core_axis_name="c", subcore_axis_name="s")

    @functools.partial(
        pl.kernel, mesh=mesh,
        out_type=(jax.ShapeDtypeStruct((_NV, _D), jnp.float32),
                  jax.ShapeDtypeStruct((_NU, _D), jnp.float32)),
        scratch_types=[
            pltpu.VMEM((_BLK, _CH), jnp.int32),
            pltpu.VMEM((_BLK, _CH), jnp.int32),
            pltpu.VMEM((_CH, _D), jnp.float32),
            pltpu.VMEM((_CH, _D), jnp.float32),
            pltpu.VMEM((_CH, _D), jnp.float32),
            pltpu.VMEM((_CH, _D), jnp.float32),
            pltpu.SemaphoreType.DMA,
            pltpu.SemaphoreType.DMA,
            pltpu.SemaphoreType.DMA,
            pltpu.SemaphoreType.DMA,
            pltpu.SemaphoreType.DMA,
            pltpu.SemaphoreType.DMA,
            pltpu.SemaphoreType.DMA,
            pltpu.SemaphoreType.DMA,
            pltpu.VMEM_SHARED((_NV, _D), jnp.float32),
        ],
    )
    def k(src_hbm, dst_hbm, tu_hbm, tv_hbm, outv_hbm, outu_hbm,
          gidx, sidx, r0, r1, r2, r3, g0, g1, g2, g3, t0, t1, t2, t3, acc):
        rows = (r0, r1, r2, r3)
        gsem = (g0, g1, g2, g3)
        tsem = (t0, t1, t2, t3)
        c = lax.axis_index("c")
        s = lax.axis_index("s")

        def _zrow(r, carry):
            for j in range(_D // 16):
                r0[r, pl.ds(j * 16, 16)] = jnp.zeros((16,), jnp.float32)
            return carry
        lax.fori_loop(0, _CH, _zrow, 0)

        def run_dir(g4d_hbm, s4d_hbm, table_hbm, out_hbm,
                    per_tile, plan, extra_plan):
            base = s * per_tile
            off = 0
            for ch in plan:
                pltpu.sync_copy(r0.at[pl.ds(0, ch)],
                                acc.at[pl.ds(base + off, ch)])
                off += ch

            @pl.when(s == 15)
            def _():
                o = per_tile
                for ch in extra_plan:
                    pltpu.sync_copy(r0.at[pl.ds(0, ch)],
                                    acc.at[pl.ds(15 * per_tile + o, ch)])
                    o += ch
            plsc.subcore_barrier()

            g4d_t = g4d_hbm.at[s]
            s4d_t = s4d_hbm.at[s]

            def block(blk, carry):
                pltpu.sync_copy(g4d_t.at[blk], gidx)
                pltpu.sync_copy(s4d_t.at[blk], sidx)
                for b in range(_NBUF):
                    pltpu.async_copy(table_hbm.at[gidx.at[b]],
                                     rows[b], gsem[b])

                def inner(io, cy):
                    ibase = io * _NBUF
                    for b in range(_NBUF):
                        i = ibase + b
                        bp = (b - 1) % _NBUF
                        pltpu.make_async_copy(
                            table_hbm.at[pl.ds(0, _CH)],
                            rows[b], gsem[b]).wait()
                        pltpu.async_copy(rows[b], acc.at[sidx.at[i]],
                                         tsem[b], add=True)
                        jg = i - 1 + _NBUF

                        @pl.when((i >= 1) & (jg < _BLK))
                        def _():
                            pltpu.make_async_copy(
                                table_hbm.at[pl.ds(0, _CH)],
                                rows[bp], tsem[bp]).wait()
                            pltpu.async_copy(table_hbm.at[gidx.at[jg]],
                                             rows[bp], gsem[bp])
                    return cy
                lax.fori_loop(0, _BLK // _NBUF, inner, 0)
                for b in range(_NBUF):
                    pltpu.make_async_copy(
                        table_hbm.at[pl.ds(0, _CH)],
                        rows[b], tsem[b]).wait()
                return carry
            lax.fori_loop(0, _NBLK, block, 0)
            plsc.subcore_barrier()

            def copy_out(row0, ch):
                pltpu.sync_copy(acc.at[pl.ds(row0, ch)],
                                r0.at[pl.ds(0, ch)])
                pltpu.sync_copy(r0.at[pl.ds(0, ch)],
                                out_hbm.at[pl.ds(row0, ch)])

            off = 0
            for ch in plan:
                copy_out(base + off, ch)
                off += ch

            @pl.when(s == 15)
            def _():
                o = per_tile
                for ch in extra_plan:
                    copy_out(15 * per_tile + o, ch)
                    o += ch

        @pl.when(c == 0)
        def _():
            run_dir(dst_hbm, src_hbm, tu_hbm, outv_hbm,
                    624, (40,) * 15 + (24,), (16,))

        @pl.when(c == 1)
        def _():
            run_dir(src_hbm, dst_hbm, tv_hbm, outu_hbm,
                    120, (40, 40, 40), (40, 40))

    return k(src4d, dst4d, table_u, table_v)


def _sc_edge_gather(lflat, src3d2, dst3d2):
    mesh = plsc.VectorSubcoreMesh(core_axis_name="c", subcore_axis_name="s")

    @functools.partial(
        pl.kernel, mesh=mesh,
        out_type=jax.ShapeDtypeStruct((32, _CPT2, _CH2), jnp.float32),
        scratch_types=[
            pltpu.VMEM((_CPT2, _CH2), jnp.int32),
            pltpu.VMEM((_CPT2, _CH2), jnp.int32),
            pltpu.VMEM((_CPT2, _CH2), jnp.int32),
            pltpu.VMEM((_CPT2, _CH2), jnp.float32),
            pltpu.SemaphoreType.DMA,
            pltpu.SemaphoreType.DMA,
            pltpu.SemaphoreType.DMA,
            pltpu.SemaphoreType.DMA,
            pltpu.SemaphoreType.DMA,
        ],
    )
    def k(lflat_hbm, src_hbm, dst_hbm, out_hbm,
          sbuf, dbuf, kbuf, lbuf, g0, g1, g2, g3, g4):
        gsem = (g0, g1, g2, g3, g4)
        c = lax.axis_index("c")
        s = lax.axis_index("s")
        w = c * _TILES + s
        pltpu.sync_copy(src_hbm.at[w], sbuf)
        pltpu.sync_copy(dst_hbm.at[w], dbuf)

        def krow(r, carry):
            for j in range(_CH2 // 16):
                sl = pl.ds(j * 16, 16)
                kbuf[r, sl] = sbuf[r, sl] * _NU + dbuf[r, sl]
            return carry
        lax.fori_loop(0, _CPT2, krow, 0)

        for b in range(_NBUF2):
            pltpu.async_copy(lflat_hbm.at[kbuf.at[b]], lbuf.at[b], gsem[b])

        def outer(io, carry):
            ibase = io * _NBUF2
            for b in range(_NBUF2):
                i = ibase + b
                pltpu.make_async_copy(
                    lflat_hbm.at[pl.ds(0, _CH2)], lbuf.at[i], gsem[b]).wait()
                nxt = i + _NBUF2

                @pl.when(nxt < _CPT2)
                def _():
                    pltpu.async_copy(
                        lflat_hbm.at[kbuf.at[nxt]], lbuf.at[nxt], gsem[b])
            return carry
        lax.fori_loop(0, _CPT2 // _NBUF2, outer, 0)
        pltpu.sync_copy(lbuf, out_hbm.at[w])

    return k(lflat, src3d2, dst3d2)


def kernel(src_v, dst_u, x_v, x_u, shuff_x_v, shuff_x_u,
           W1v, W1u, b1v, b1u, alpha1, W2v, W2u, b2v, b2u, Wb):
    src4d = src_v.reshape(_TILES, _NBLK, _BLK, _CH)
    dst4d = dst_u.reshape(_TILES, _NBLK, _BLK, _CH)
    src3d2 = src_v.reshape(32, _CPT2, _CH2)
    dst3d2 = dst_u.reshape(32, _CPT2, _CH2)
    b1v_r = b1v.reshape(1, _D)
    b1u_r = b1u.reshape(1, _D)
    b2v_r = b2v.reshape(1, _D)
    b2u_r = b2u.reshape(1, _D)
    al_r = jnp.broadcast_to(alpha1, (1, _D)).astype(jnp.float32)
    wv_cat = jnp.concatenate([W1v, W2v[_D:, :]], axis=1)
    wu_cat = jnp.concatenate([W1u, W2u[_D:, :]], axis=1)

    hv_cat, hu_cat = _tc_mm_pair(x_v, x_u, wv_cat, wu_cat)
    hv, gxv = hv_cat[:, :_D], hv_cat[:, _D:]
    hu, gxu = hu_cat[:, :_D], hu_cat[:, _D:]

    sv1, su1 = _sc_segsum_pair(src4d, dst4d, hu, hv)

    gv, gu = _tc_layer2(sv1, su1, gxv, gxu, W2v[:_D, :], W2u[:_D, :],
                        b1v_r, b1u_r, al_r)

    sv2, su2 = _sc_segsum_pair(src4d, dst4d, gu, gv)

    ve2, ue2, av = _tc_disc_prep(sv2, su2, b2v_r, b2u_r, Wb)

    big_l, s0 = _tc_logits(av, ue2)

    l2d = _sc_edge_gather(big_l.reshape(-1), src3d2, dst3d2)
    sneg, spos = _tc_edge_reduce(l2d.reshape(_E // _D, _D))

    n = float(_NV) * float(_NU)
    tsum = float(_E)
    pos_weight = (n - tsum) / tsum
    norm = n / (n - tsum)
    sum_bce = -s0[0, 0] + sneg[0, 0] - pos_weight * spos[0, 0]
    loss = norm * sum_bce / n
    return ve2, ue2, loss


# speedup vs baseline: 9.6749x; 1.0167x over previous
import functools

import jax
import jax.numpy as jnp
from jax import lax
from jax.experimental import pallas as pl
from jax.experimental.pallas import tpu as pltpu
from jax.experimental.pallas import tpu_sc as plsc

_NV, _NU, _E = 10000, 2000, 320000
_D = 128
_CH = 40
_NBLK = 10
_BLK = 50
_NBUF = 5
_TILES = 16
_CH2 = 80
_CPT2 = _E // _CH2 // 32
_NBUF2 = 5


def _mm_pair_body(xv_ref, xu_ref, wv_ref, wu_ref, ov_ref, ou_ref):
    ov_ref[...] = jnp.dot(xv_ref[...], wv_ref[...],
                          preferred_element_type=jnp.float32)
    ou_ref[...] = jnp.dot(xu_ref[...], wu_ref[...],
                          preferred_element_type=jnp.float32)


def _tc_mm_pair(x_v, x_u, wv, wu):
    return pl.pallas_call(
        _mm_pair_body,
        out_shape=(jax.ShapeDtypeStruct((_NV, 2 * _D), jnp.float32),
                   jax.ShapeDtypeStruct((_NU, 2 * _D), jnp.float32)),
    )(x_v, x_u, wv, wu)


def _layer2_body(sv_ref, su_ref, gxv_ref, gxu_ref, wv_ref, wu_ref,
                 bv_ref, bu_ref, al_ref, gv_ref, gu_ref):
    al = al_ref[...]
    av = sv_ref[0:_NV, :] + bv_ref[...]
    av = jnp.where(av >= 0.0, av, al * av)
    gv_ref[...] = jnp.dot(av, wv_ref[...],
                          preferred_element_type=jnp.float32) + gxv_ref[...]
    au = su_ref[0:_NU, :] + bu_ref[...]
    au = jnp.where(au >= 0.0, au, al * au)
    gu_ref[...] = jnp.dot(au, wu_ref[...],
                          preferred_element_type=jnp.float32) + gxu_ref[...]


def _tc_layer2(sv1, su1, gxv, gxu, w2v_top, w2u_top, b1v, b1u, alpha):
    return pl.pallas_call(
        _layer2_body,
        out_shape=(jax.ShapeDtypeStruct((_NV, _D), jnp.float32),
                   jax.ShapeDtypeStruct((_NU, _D), jnp.float32)),
    )(sv1, su1, gxv, gxu, w2v_top, w2u_top, b1v, b1u, alpha)


def _disc_prep_body(sv_ref, su_ref, bv_ref, bu_ref, wb_ref,
                    ve2_ref, ue2_ref, av_ref):
    v = sv_ref[0:_NV, :] + bv_ref[...]
    ve2_ref[...] = v
    ue2_ref[...] = su_ref[0:_NU, :] + bu_ref[...]
    av_ref[...] = jnp.dot(v, wb_ref[...], preferred_element_type=jnp.float32)


def _tc_disc_prep(sv2, su2, b2v, b2u, wb):
    return pl.pallas_call(
        _disc_prep_body,
        out_shape=(jax.ShapeDtypeStruct((_NV, _D), jnp.float32),
                   jax.ShapeDtypeStruct((_NU, _D), jnp.float32),
                   jax.ShapeDtypeStruct((_NV, _D), jnp.float32)),
    )(sv2, su2, b2v, b2u, wb)


def _ls(x):
    return jnp.minimum(x, 0.0) - jnp.log(1.0 + jnp.exp(-jnp.abs(x)))


_BM = 400


def _logits_body(av_ref, u_ref, l_ref, s0_ref):
    band = lax.dot_general(av_ref[...], u_ref[...],
                           (((1,), (1,)), ((), ())),
                           preferred_element_type=jnp.float32)
    l_ref[...] = band
    part = jnp.sum(_ls(-band))
    pid = pl.program_id(0)
    s0_ref[...] = jnp.where(pid == 0, part, s0_ref[...] + part)


def _tc_logits(av, ue2):
    nb = _NV // _BM
    return pl.pallas_call(
        _logits_body,
        grid=(nb,),
        in_specs=[pl.BlockSpec((_BM, _D), lambda i: (i, 0)),
                  pl.BlockSpec((_NU, _D), lambda i: (0, 0))],
        out_specs=(pl.BlockSpec((_BM, _NU), lambda i: (i, 0)),
                   pl.BlockSpec((1, 1), lambda i: (0, 0))),
        out_shape=(jax.ShapeDtypeStruct((_NV, _NU), jnp.float32),
                   jax.ShapeDtypeStruct((1, 1), jnp.float32)),
    )(av, ue2)


def _edge_reduce_body(l_ref, sneg_ref, spos_ref):
    x = l_ref[...]
    sneg_ref[...] = jnp.sum(_ls(-x)).reshape(1, 1)
    spos_ref[...] = jnp.sum(_ls(x)).reshape(1, 1)


def _tc_edge_reduce(l2d):
    return pl.pallas_call(
        _edge_reduce_body,
        out_shape=(jax.ShapeDtypeStruct((1, 1), jnp.float32),
                   jax.ShapeDtypeStruct((1, 1), jnp.float32)),
    )(l2d)


def _sc_segsum_pair(src4d, dst4d, table_u, table_v):
    mesh = plsc.VectorSubcoreMesh(core_axis_name="c", subcore_axis_name="s")

    @functools.partial(
        pl.kernel, mesh=mesh,
        out_type=(jax.ShapeDtypeStruct((_NV, _D), jnp.float32),
                  jax.ShapeDtypeStruct((_NU, _D), jnp.float32)),
        scratch_types=[
            pltpu.VMEM((_BLK, _CH), jnp.int32),
            pltpu.VMEM((_BLK, _CH), jnp.int32),
            pltpu.VMEM((_CH, _D), jnp.float32),
            pltpu.VMEM((_CH, _D), jnp.float32),
            pltpu.VMEM((_CH, _D), jnp.float32),
            pltpu.VMEM((_CH, _D), jnp.float32),
            pltpu.VMEM((_CH, _D), jnp.float32),
            pltpu.SemaphoreType.DMA,
            pltpu.SemaphoreType.DMA,
            pltpu.SemaphoreType.DMA,
            pltpu.SemaphoreType.DMA,
            pltpu.SemaphoreType.DMA,
            pltpu.SemaphoreType.DMA,
            pltpu.SemaphoreType.DMA,
            pltpu.SemaphoreType.DMA,
            pltpu.SemaphoreType.DMA,
            pltpu.SemaphoreType.DMA,
            pltpu.VMEM_SHARED((_NV, _D), jnp.float32),
        ],
    )
    def k(src_hbm, dst_hbm, tu_hbm, tv_hbm, outv_hbm, outu_hbm,
          gidx, sidx, r0, r1, r2, r3, r4,
          g0, g1, g2, g3, g4, t0, t1, t2, t3, t4, acc):
        rows = (r0, r1, r2, r3, r4)
        gsem = (g0, g1, g2, g3, g4)
        tsem = (t0, t1, t2, t3, t4)
        c = lax.axis_index("c")
        s = lax.axis_index("s")

        def _zrow(r, carry):
            for j in range(_D // 16):
                r0[r, pl.ds(j * 16, 16)] = jnp.zeros((16,), jnp.float32)
            return carry
        lax.fori_loop(0, _CH, _zrow, 0)

        def run_dir(g4d_hbm, s4d_hbm, table_hbm, out_hbm,
                    per_tile, plan, extra_plan):
            base = s * per_tile
            off = 0
            for ch in plan:
                pltpu.sync_copy(r0.at[pl.ds(0, ch)],
                                acc.at[pl.ds(base + off, ch)])
                off += ch

            @pl.when(s == 15)
            def _():
                o = per_tile
                for ch in extra_plan:
                    pltpu.sync_copy(r0.at[pl.ds(0, ch)],
                                    acc.at[pl.ds(15 * per_tile + o, ch)])
                    o += ch
            plsc.subcore_barrier()

            g4d_t = g4d_hbm.at[s]
            s4d_t = s4d_hbm.at[s]

            def block(blk, carry):
                pltpu.sync_copy(g4d_t.at[blk], gidx)
                pltpu.sync_copy(s4d_t.at[blk], sidx)
                for b in range(_NBUF):
                    pltpu.async_copy(table_hbm.at[gidx.at[b]],
                                     rows[b], gsem[b])

                def inner(io, cy):
                    ibase = io * _NBUF
                    for b in range(_NBUF):
                        i = ibase + b
                        bp = (b - 1) % _NBUF
                        pltpu.make_async_copy(
                            table_hbm.at[pl.ds(0, _CH)],
                            rows[b], gsem[b]).wait()
                        pltpu.async_copy(rows[b], acc.at[sidx.at[i]],
                                         tsem[b], add=True)
                        jg = i - 1 + _NBUF

                        @pl.when((i >= 1) & (jg < _BLK))
                        def _():
                            pltpu.make_async_copy(
                                table_hbm.at[pl.ds(0, _CH)],
                                rows[bp], tsem[bp]).wait()
                            pltpu.async_copy(table_hbm.at[gidx.at[jg]],
                                             rows[bp], gsem[bp])
                    return cy
                lax.fori_loop(0, _BLK // _NBUF, inner, 0)
                for b in range(_NBUF):
                    pltpu.make_async_copy(
                        table_hbm.at[pl.ds(0, _CH)],
                        rows[b], tsem[b]).wait()
                return carry
            lax.fori_loop(0, _NBLK, block, 0)
            plsc.subcore_barrier()

            def copy_out(row0, ch):
                pltpu.sync_copy(acc.at[pl.ds(row0, ch)],
                                r0.at[pl.ds(0, ch)])
                pltpu.sync_copy(r0.at[pl.ds(0, ch)],
                                out_hbm.at[pl.ds(row0, ch)])

            off = 0
            for ch in plan:
                copy_out(base + off, ch)
                off += ch

            @pl.when(s == 15)
            def _():
                o = per_tile
                for ch in extra_plan:
                    copy_out(15 * per_tile + o, ch)
                    o += ch

        @pl.when(c == 0)
        def _():
            run_dir(dst_hbm, src_hbm, tu_hbm, outv_hbm,
                    624, (40,) * 15 + (24,), (16,))

        @pl.when(c == 1)
        def _():
            run_dir(src_hbm, dst_hbm, tv_hbm, outu_hbm,
                    120, (40, 40, 40), (40, 40))

    return k(src4d, dst4d, table_u, table_v)


def _sc_edge_gather(lflat, src3d2, dst3d2):
    mesh = plsc.VectorSubcoreMesh(core_axis_name="c", subcore_axis_name="s")

    @functools.partial(
        pl.kernel, mesh=mesh,
        out_type=jax.ShapeDtypeStruct((32, _CPT2, _CH2), jnp.float32),
        scratch_types=[
            pltpu.VMEM((_CPT2, _CH2), jnp.int32),
            pltpu.VMEM((_CPT2, _CH2), jnp.int32),
            pltpu.VMEM((_CPT2, _CH2), jnp.int32),
            pltpu.VMEM((_CPT2, _CH2), jnp.float32),
            pltpu.SemaphoreType.DMA,
            pltpu.SemaphoreType.DMA,
            pltpu.SemaphoreType.DMA,
            pltpu.SemaphoreType.DMA,
            pltpu.SemaphoreType.DMA,
        ],
    )
    def k(lflat_hbm, src_hbm, dst_hbm, out_hbm,
          sbuf, dbuf, kbuf, lbuf, g0, g1, g2, g3, g4):
        gsem = (g0, g1, g2, g3, g4)
        c = lax.axis_index("c")
        s = lax.axis_index("s")
        w = c * _TILES + s
        pltpu.sync_copy(src_hbm.at[w], sbuf)
        pltpu.sync_copy(dst_hbm.at[w], dbuf)

        def krow(r, carry):
            for j in range(_CH2 // 16):
                sl = pl.ds(j * 16, 16)
                kbuf[r, sl] = sbuf[r, sl] * _NU + dbuf[r, sl]
            return carry
        lax.fori_loop(0, _CPT2, krow, 0)

        for b in range(_NBUF2):
            pltpu.async_copy(lflat_hbm.at[kbuf.at[b]], lbuf.at[b], gsem[b])

        def outer(io, carry):
            ibase = io * _NBUF2
            for b in range(_NBUF2):
                i = ibase + b
                pltpu.make_async_copy(
                    lflat_hbm.at[pl.ds(0, _CH2)], lbuf.at[i], gsem[b]).wait()
                nxt = i + _NBUF2

                @pl.when(nxt < _CPT2)
                def _():
                    pltpu.async_copy(
                        lflat_hbm.at[kbuf.at[nxt]], lbuf.at[nxt], gsem[b])
            return carry
        lax.fori_loop(0, _CPT2 // _NBUF2, outer, 0)
        pltpu.sync_copy(lbuf, out_hbm.at[w])

    return k(lflat, src3d2, dst3d2)


def kernel(src_v, dst_u, x_v, x_u, shuff_x_v, shuff_x_u,
           W1v, W1u, b1v, b1u, alpha1, W2v, W2u, b2v, b2u, Wb):
    src4d = src_v.reshape(_TILES, _NBLK, _BLK, _CH)
    dst4d = dst_u.reshape(_TILES, _NBLK, _BLK, _CH)
    src3d2 = src_v.reshape(32, _CPT2, _CH2)
    dst3d2 = dst_u.reshape(32, _CPT2, _CH2)
    b1v_r = b1v.reshape(1, _D)
    b1u_r = b1u.reshape(1, _D)
    b2v_r = b2v.reshape(1, _D)
    b2u_r = b2u.reshape(1, _D)
    al_r = jnp.broadcast_to(alpha1, (1, _D)).astype(jnp.float32)
    wv_cat = jnp.concatenate([W1v, W2v[_D:, :]], axis=1)
    wu_cat = jnp.concatenate([W1u, W2u[_D:, :]], axis=1)

    hv_cat, hu_cat = _tc_mm_pair(x_v, x_u, wv_cat, wu_cat)
    hv, gxv = hv_cat[:, :_D], hv_cat[:, _D:]
    hu, gxu = hu_cat[:, :_D], hu_cat[:, _D:]

    sv1, su1 = _sc_segsum_pair(src4d, dst4d, hu, hv)

    gv, gu = _tc_layer2(sv1, su1, gxv, gxu, W2v[:_D, :], W2u[:_D, :],
                        b1v_r, b1u_r, al_r)

    sv2, su2 = _sc_segsum_pair(src4d, dst4d, gu, gv)

    ve2, ue2, av = _tc_disc_prep(sv2, su2, b2v_r, b2u_r, Wb)

    big_l, s0 = _tc_logits(av, ue2)

    l2d = _sc_edge_gather(big_l.reshape(-1), src3d2, dst3d2)
    sneg, spos = _tc_edge_reduce(l2d.reshape(_E // _D, _D))

    n = float(_NV) * float(_NU)
    tsum = float(_E)
    pos_weight = (n - tsum) / tsum
    norm = n / (n - tsum)
    sum_bce = -s0[0, 0] + sneg[0, 0] - pos_weight * spos[0, 0]
    loss = norm * sum_bce / n
    return ve2, ue2, loss


# speedup vs baseline: 9.7356x; 1.0063x over previous
import functools

import jax
import jax.numpy as jnp
from jax import lax
from jax.experimental import pallas as pl
from jax.experimental.pallas import tpu as pltpu
from jax.experimental.pallas import tpu_sc as plsc

_NV, _NU, _E = 10000, 2000, 320000
_D = 128
_CH = 40
_NBLK = 10
_BLK = 50
_NBUF = 5
_TILES = 16
_CH2 = 80
_CPT2 = _E // _CH2 // 32
_NBUF2 = 5


def _mm_pair_body(xv_ref, xu_ref, wv_ref, wu_ref, ov_ref, ou_ref):
    ov_ref[...] = jnp.dot(xv_ref[...], wv_ref[...],
                          preferred_element_type=jnp.float32)
    ou_ref[...] = jnp.dot(xu_ref[...], wu_ref[...],
                          preferred_element_type=jnp.float32)


def _tc_mm_pair(x_v, x_u, wv, wu):
    return pl.pallas_call(
        _mm_pair_body,
        out_shape=(jax.ShapeDtypeStruct((_NV, 2 * _D), jnp.float32),
                   jax.ShapeDtypeStruct((_NU, 2 * _D), jnp.float32)),
    )(x_v, x_u, wv, wu)


def _layer2_body(sv_ref, su_ref, gxv_ref, gxu_ref, wv_ref, wu_ref,
                 bv_ref, bu_ref, al_ref, gv_ref, gu_ref):
    al = al_ref[...]
    av = sv_ref[0:_NV, :] + bv_ref[...]
    av = jnp.where(av >= 0.0, av, al * av)
    gv_ref[...] = jnp.dot(av, wv_ref[...],
                          preferred_element_type=jnp.float32) + gxv_ref[...]
    au = su_ref[0:_NU, :] + bu_ref[...]
    au = jnp.where(au >= 0.0, au, al * au)
    gu_ref[...] = jnp.dot(au, wu_ref[...],
                          preferred_element_type=jnp.float32) + gxu_ref[...]


def _tc_layer2(sv1, su1, gxv, gxu, w2v_top, w2u_top, b1v, b1u, alpha):
    return pl.pallas_call(
        _layer2_body,
        out_shape=(jax.ShapeDtypeStruct((_NV, _D), jnp.float32),
                   jax.ShapeDtypeStruct((_NU, _D), jnp.float32)),
    )(sv1, su1, gxv, gxu, w2v_top, w2u_top, b1v, b1u, alpha)


def _ls(x):
    return jnp.minimum(x, 0.0) - jnp.log(1.0 + jnp.exp(-jnp.abs(x)))


_BM = 400


def _disc_logits_body(sv_ref, su_ref, bv_ref, bu_ref, wb_ref,
                      ve2_ref, ue2_ref, l_ref, s0_ref):
    pid = pl.program_id(0)
    v = sv_ref[...] + bv_ref[...]
    ve2_ref[...] = v
    u = su_ref[...] + bu_ref[...]

    @pl.when(pid == 0)
    def _():
        ue2_ref[...] = u
    av = jnp.dot(v, wb_ref[...], preferred_element_type=jnp.float32)
    band = lax.dot_general(av, u, (((1,), (1,)), ((), ())),
                           preferred_element_type=jnp.float32)
    l_ref[...] = band
    part = jnp.sum(_ls(-band))
    s0_ref[...] = jnp.where(pid == 0, part, s0_ref[...] + part)


def _tc_disc_logits(sv2, su2, b2v, b2u, wb):
    nb = _NV // _BM
    return pl.pallas_call(
        _disc_logits_body,
        grid=(nb,),
        in_specs=[pl.BlockSpec((_BM, _D), lambda i: (i, 0)),
                  pl.BlockSpec((_NU, _D), lambda i: (0, 0)),
                  pl.BlockSpec((1, _D), lambda i: (0, 0)),
                  pl.BlockSpec((1, _D), lambda i: (0, 0)),
                  pl.BlockSpec((_D, _D), lambda i: (0, 0))],
        out_specs=(pl.BlockSpec((_BM, _D), lambda i: (i, 0)),
                   pl.BlockSpec((_NU, _D), lambda i: (0, 0)),
                   pl.BlockSpec((_BM, _NU), lambda i: (i, 0)),
                   pl.BlockSpec((1, 1), lambda i: (0, 0))),
        out_shape=(jax.ShapeDtypeStruct((_NV, _D), jnp.float32),
                   jax.ShapeDtypeStruct((_NU, _D), jnp.float32),
                   jax.ShapeDtypeStruct((_NV, _NU), jnp.float32),
                   jax.ShapeDtypeStruct((1, 1), jnp.float32)),
    )(sv2, su2, b2v, b2u, wb)


def _edge_reduce_body(l_ref, sneg_ref, spos_ref):
    x = l_ref[...]
    sneg_ref[...] = jnp.sum(_ls(-x)).reshape(1, 1)
    spos_ref[...] = jnp.sum(_ls(x)).reshape(1, 1)


def _tc_edge_reduce(l2d):
    return pl.pallas_call(
        _edge_reduce_body,
        out_shape=(jax.ShapeDtypeStruct((1, 1), jnp.float32),
                   jax.ShapeDtypeStruct((1, 1), jnp.float32)),
    )(l2d)


def _sc_segsum_pair(src4d, dst4d, table_u, table_v):
    mesh = plsc.VectorSubcoreMesh(core_axis_name="c", subcore_axis_name="s")

    @functools.partial(
        pl.kernel, mesh=mesh,
        out_type=(jax.ShapeDtypeStruct((_NV, _D), jnp.float32),
                  jax.ShapeDtypeStruct((_NU, _D), jnp.float32)),
        scratch_types=[
            pltpu.VMEM((_BLK, _CH), jnp.int32),
            pltpu.VMEM((_BLK, _CH), jnp.int32),
            pltpu.VMEM((_CH, _D), jnp.float32),
            pltpu.VMEM((_CH, _D), jnp.float32),
            pltpu.VMEM((_CH, _D), jnp.float32),
            pltpu.VMEM((_CH, _D), jnp.float32),
            pltpu.VMEM((_CH, _D), jnp.float32),
            pltpu.SemaphoreType.DMA,
            pltpu.SemaphoreType.DMA,
            pltpu.SemaphoreType.DMA,
            pltpu.SemaphoreType.DMA,
            pltpu.SemaphoreType.DMA,
            pltpu.SemaphoreType.DMA,
            pltpu.SemaphoreType.DMA,
            pltpu.SemaphoreType.DMA,
            pltpu.SemaphoreType.DMA,
            pltpu.SemaphoreType.DMA,
            pltpu.VMEM_SHARED((_NV, _D), jnp.float32),
        ],
    )
    def k(src_hbm, dst_hbm, tu_hbm, tv_hbm, outv_hbm, outu_hbm,
          gidx, sidx, r0, r1, r2, r3, r4,
          g0, g1, g2, g3, g4, t0, t1, t2, t3, t4, acc):
        rows = (r0, r1, r2, r3, r4)
        gsem = (g0, g1, g2, g3, g4)
        tsem = (t0, t1, t2, t3, t4)
        c = lax.axis_index("c")
        s = lax.axis_index("s")

        def _zrow(r, carry):
            for j in range(_D // 16):
                r0[r, pl.ds(j * 16, 16)] = jnp.zeros((16,), jnp.float32)
            return carry
        lax.fori_loop(0, _CH, _zrow, 0)

        def run_dir(g4d_hbm, s4d_hbm, table_hbm, out_hbm,
                    per_tile, plan, extra_plan):
            base = s * per_tile
            off = 0
            for ch in plan:
                pltpu.sync_copy(r0.at[pl.ds(0, ch)],
                                acc.at[pl.ds(base + off, ch)])
                off += ch

            @pl.when(s == 15)
            def _():
                o = per_tile
                for ch in extra_plan:
                    pltpu.sync_copy(r0.at[pl.ds(0, ch)],
                                    acc.at[pl.ds(15 * per_tile + o, ch)])
                    o += ch
            plsc.subcore_barrier()

            g4d_t = g4d_hbm.at[s]
            s4d_t = s4d_hbm.at[s]

            def block(blk, carry):
                pltpu.sync_copy(g4d_t.at[blk], gidx)
                pltpu.sync_copy(s4d_t.at[blk], sidx)
                for b in range(_NBUF):
                    pltpu.async_copy(table_hbm.at[gidx.at[b]],
                                     rows[b], gsem[b])

                def inner(io, cy):
                    ibase = io * _NBUF
                    for b in range(_NBUF):
                        i = ibase + b
                        bp = (b - 1) % _NBUF
                        pltpu.make_async_copy(
                            table_hbm.at[pl.ds(0, _CH)],
                            rows[b], gsem[b]).wait()
                        pltpu.async_copy(rows[b], acc.at[sidx.at[i]],
                                         tsem[b], add=True)
                        jg = i - 1 + _NBUF

                        @pl.when((i >= 1) & (jg < _BLK))
                        def _():
                            pltpu.make_async_copy(
                                table_hbm.at[pl.ds(0, _CH)],
                                rows[bp], tsem[bp]).wait()
                            pltpu.async_copy(table_hbm.at[gidx.at[jg]],
                                             rows[bp], gsem[bp])
                    return cy
                lax.fori_loop(0, _BLK // _NBUF, inner, 0)
                for b in range(_NBUF):
                    pltpu.make_async_copy(
                        table_hbm.at[pl.ds(0, _CH)],
                        rows[b], tsem[b]).wait()
                return carry
            lax.fori_loop(0, _NBLK, block, 0)
            plsc.subcore_barrier()

            def copy_out(row0, ch):
                pltpu.sync_copy(acc.at[pl.ds(row0, ch)],
                                r0.at[pl.ds(0, ch)])
                pltpu.sync_copy(r0.at[pl.ds(0, ch)],
                                out_hbm.at[pl.ds(row0, ch)])

            off = 0
            for ch in plan:
                copy_out(base + off, ch)
                off += ch

            @pl.when(s == 15)
            def _():
                o = per_tile
                for ch in extra_plan:
                    copy_out(15 * per_tile + o, ch)
                    o += ch

        @pl.when(c == 0)
        def _():
            run_dir(dst_hbm, src_hbm, tu_hbm, outv_hbm,
                    624, (40,) * 15 + (24,), (16,))

        @pl.when(c == 1)
        def _():
            run_dir(src_hbm, dst_hbm, tv_hbm, outu_hbm,
                    120, (40, 40, 40), (40, 40))

    return k(src4d, dst4d, table_u, table_v)


def _sc_edge_gather(lflat, src3d2, dst3d2):
    mesh = plsc.VectorSubcoreMesh(core_axis_name="c", subcore_axis_name="s")

    @functools.partial(
        pl.kernel, mesh=mesh,
        out_type=jax.ShapeDtypeStruct((32, _CPT2, _CH2), jnp.float32),
        scratch_types=[
            pltpu.VMEM((_CPT2, _CH2), jnp.int32),
            pltpu.VMEM((_CPT2, _CH2), jnp.int32),
            pltpu.VMEM((_CPT2, _CH2), jnp.int32),
            pltpu.VMEM((_CPT2, _CH2), jnp.float32),
            pltpu.SemaphoreType.DMA,
            pltpu.SemaphoreType.DMA,
            pltpu.SemaphoreType.DMA,
            pltpu.SemaphoreType.DMA,
            pltpu.SemaphoreType.DMA,
        ],
    )
    def k(lflat_hbm, src_hbm, dst_hbm, out_hbm,
          sbuf, dbuf, kbuf, lbuf, g0, g1, g2, g3, g4):
        gsem = (g0, g1, g2, g3, g4)
        c = lax.axis_index("c")
        s = lax.axis_index("s")
        w = c * _TILES + s
        pltpu.sync_copy(src_hbm.at[w], sbuf)
        pltpu.sync_copy(dst_hbm.at[w], dbuf)

        def krow(r, carry):
            for j in range(_CH2 // 16):
                sl = pl.ds(j * 16, 16)
                kbuf[r, sl] = sbuf[r, sl] * _NU + dbuf[r, sl]
            return carry
        lax.fori_loop(0, _CPT2, krow, 0)

        for b in range(_NBUF2):
            pltpu.async_copy(lflat_hbm.at[kbuf.at[b]], lbuf.at[b], gsem[b])

        def outer(io, carry):
            ibase = io * _NBUF2
            for b in range(_NBUF2):
                i = ibase + b
                pltpu.make_async_copy(
                    lflat_hbm.at[pl.ds(0, _CH2)], lbuf.at[i], gsem[b]).wait()
                nxt = i + _NBUF2

                @pl.when(nxt < _CPT2)
                def _():
                    pltpu.async_copy(
                        lflat_hbm.at[kbuf.at[nxt]], lbuf.at[nxt], gsem[b])
            return carry
        lax.fori_loop(0, _CPT2 // _NBUF2, outer, 0)
        pltpu.sync_copy(lbuf, out_hbm.at[w])

    return k(lflat, src3d2, dst3d2)


def kernel(src_v, dst_u, x_v, x_u, shuff_x_v, shuff_x_u,
           W1v, W1u, b1v, b1u, alpha1, W2v, W2u, b2v, b2u, Wb):
    src4d = src_v.reshape(_TILES, _NBLK, _BLK, _CH)
    dst4d = dst_u.reshape(_TILES, _NBLK, _BLK, _CH)
    src3d2 = src_v.reshape(32, _CPT2, _CH2)
    dst3d2 = dst_u.reshape(32, _CPT2, _CH2)
    b1v_r = b1v.reshape(1, _D)
    b1u_r = b1u.reshape(1, _D)
    b2v_r = b2v.reshape(1, _D)
    b2u_r = b2u.reshape(1, _D)
    al_r = jnp.broadcast_to(alpha1, (1, _D)).astype(jnp.float32)
    wv_cat = jnp.concatenate([W1v, W2v[_D:, :]], axis=1)
    wu_cat = jnp.concatenate([W1u, W2u[_D:, :]], axis=1)

    hv_cat, hu_cat = _tc_mm_pair(x_v, x_u, wv_cat, wu_cat)
    hv, gxv = hv_cat[:, :_D], hv_cat[:, _D:]
    hu, gxu = hu_cat[:, :_D], hu_cat[:, _D:]

    sv1, su1 = _sc_segsum_pair(src4d, dst4d, hu, hv)

    gv, gu = _tc_layer2(sv1, su1, gxv, gxu, W2v[:_D, :], W2u[:_D, :],
                        b1v_r, b1u_r, al_r)

    sv2, su2 = _sc_segsum_pair(src4d, dst4d, gu, gv)

    ve2, ue2, big_l, s0 = _tc_disc_logits(sv2, su2, b2v_r, b2u_r, Wb)

    l2d = _sc_edge_gather(big_l.reshape(-1), src3d2, dst3d2)
    sneg, spos = _tc_edge_reduce(l2d.reshape(_E // _D, _D))

    n = float(_NV) * float(_NU)
    tsum = float(_E)
    pos_weight = (n - tsum) / tsum
    norm = n / (n - tsum)
    sum_bce = -s0[0, 0] + sneg[0, 0] - pos_weight * spos[0, 0]
    loss = norm * sum_bce / n
    return ve2, ue2, loss


# speedup vs baseline: 10.1910x; 1.0468x over previous
import functools

import jax
import jax.numpy as jnp
from jax import lax
from jax.experimental import pallas as pl
from jax.experimental.pallas import tpu as pltpu
from jax.experimental.pallas import tpu_sc as plsc

_NV, _NU, _E = 10000, 2000, 320000
_D = 128
_CH = 40
_NBLK = 10
_BLK = 50
_NBUF = 5
_TILES = 16
_CH2 = 80
_CPT2 = _E // _CH2 // 32
_NBUF2 = 5


def _mm_pair_body(xv_ref, xu_ref, wv_ref, wu_ref, ov_ref, ou_ref):
    ov_ref[...] = jnp.dot(xv_ref[...], wv_ref[...],
                          preferred_element_type=jnp.float32)
    ou_ref[...] = jnp.dot(xu_ref[...], wu_ref[...],
                          preferred_element_type=jnp.float32)


def _tc_mm_pair(x_v, x_u, wv, wu):
    return pl.pallas_call(
        _mm_pair_body,
        out_shape=(jax.ShapeDtypeStruct((_NV, 2 * _D), jnp.float32),
                   jax.ShapeDtypeStruct((_NU, 2 * _D), jnp.float32)),
    )(x_v, x_u, wv, wu)


def _layer2_body(sv_ref, su_ref, gxv_ref, gxu_ref, wv_ref, wu_ref,
                 bv_ref, bu_ref, al_ref, gv_ref, gu_ref):
    al = al_ref[...]
    av = sv_ref[0:_NV, :] + bv_ref[...]
    av = jnp.where(av >= 0.0, av, al * av)
    gv_ref[...] = jnp.dot(av, wv_ref[...],
                          preferred_element_type=jnp.float32) + gxv_ref[...]
    au = su_ref[0:_NU, :] + bu_ref[...]
    au = jnp.where(au >= 0.0, au, al * au)
    gu_ref[...] = jnp.dot(au, wu_ref[...],
                          preferred_element_type=jnp.float32) + gxu_ref[...]


def _tc_layer2(sv1, su1, gxv, gxu, w2v_top, w2u_top, b1v, b1u, alpha):
    return pl.pallas_call(
        _layer2_body,
        out_shape=(jax.ShapeDtypeStruct((_NV, _D), jnp.float32),
                   jax.ShapeDtypeStruct((_NU, _D), jnp.float32)),
    )(sv1, su1, gxv, gxu, w2v_top, w2u_top, b1v, b1u, alpha)


def _ls(x):
    return jnp.minimum(x, 0.0) - jnp.log(1.0 + jnp.exp(-jnp.abs(x)))


_BM = 400
_NUP = 2048


def _disc_logits_body(sv_ref, su_ref, bv_ref, bu_ref, wb_ref,
                      ve2_ref, ue2_ref, l_ref, s0_ref):
    pid = pl.program_id(0)
    v = sv_ref[...] + bv_ref[...]
    ve2_ref[...] = v
    rid = lax.broadcasted_iota(jnp.int32, (_NUP, _D), 0)
    u = jnp.where(rid < _NU, su_ref[...] + bu_ref[...], 0.0)

    @pl.when(pid == 0)
    def _():
        ue2_ref[...] = u[0:_NU, :]
    av = jnp.dot(v, wb_ref[...], preferred_element_type=jnp.float32)
    band = lax.dot_general(av, u, (((1,), (1,)), ((), ())),
                           preferred_element_type=jnp.float32)
    l_ref[...] = band
    part = jnp.sum(_ls(-band))
    s0_ref[...] = jnp.where(pid == 0, part, s0_ref[...] + part)


def _tc_disc_logits(sv2, su2p, b2v, b2u, wb):
    nb = _NV // _BM
    return pl.pallas_call(
        _disc_logits_body,
        grid=(nb,),
        in_specs=[pl.BlockSpec((_BM, _D), lambda i: (i, 0)),
                  pl.BlockSpec((_NUP, _D), lambda i: (0, 0)),
                  pl.BlockSpec((1, _D), lambda i: (0, 0)),
                  pl.BlockSpec((1, _D), lambda i: (0, 0)),
                  pl.BlockSpec((_D, _D), lambda i: (0, 0))],
        out_specs=(pl.BlockSpec((_BM, _D), lambda i: (i, 0)),
                   pl.BlockSpec((_NU, _D), lambda i: (0, 0)),
                   pl.BlockSpec((_BM, _NUP), lambda i: (i, 0)),
                   pl.BlockSpec((1, 1), lambda i: (0, 0))),
        out_shape=(jax.ShapeDtypeStruct((_NV, _D), jnp.float32),
                   jax.ShapeDtypeStruct((_NU, _D), jnp.float32),
                   jax.ShapeDtypeStruct((_NV, _NUP), jnp.float32),
                   jax.ShapeDtypeStruct((1, 1), jnp.float32)),
    )(sv2, su2p, b2v, b2u, wb)


def _edge_reduce_body(l_ref, sneg_ref, spos_ref):
    x = l_ref[...]
    sneg_ref[...] = jnp.sum(_ls(-x)).reshape(1, 1)
    spos_ref[...] = jnp.sum(_ls(x)).reshape(1, 1)


def _tc_edge_reduce(l2d):
    return pl.pallas_call(
        _edge_reduce_body,
        out_shape=(jax.ShapeDtypeStruct((1, 1), jnp.float32),
                   jax.ShapeDtypeStruct((1, 1), jnp.float32)),
    )(l2d)


def _sc_segsum_pair(src4d, dst4d, table_u, table_v):
    mesh = plsc.VectorSubcoreMesh(core_axis_name="c", subcore_axis_name="s")

    @functools.partial(
        pl.kernel, mesh=mesh,
        out_type=(jax.ShapeDtypeStruct((_NV, _D), jnp.float32),
                  jax.ShapeDtypeStruct((_NU, _D), jnp.float32)),
        scratch_types=[
            pltpu.VMEM((_BLK, _CH), jnp.int32),
            pltpu.VMEM((_BLK, _CH), jnp.int32),
            pltpu.VMEM((_CH, _D), jnp.float32),
            pltpu.VMEM((_CH, _D), jnp.float32),
            pltpu.VMEM((_CH, _D), jnp.float32),
            pltpu.VMEM((_CH, _D), jnp.float32),
            pltpu.VMEM((_CH, _D), jnp.float32),
            pltpu.SemaphoreType.DMA,
            pltpu.SemaphoreType.DMA,
            pltpu.SemaphoreType.DMA,
            pltpu.SemaphoreType.DMA,
            pltpu.SemaphoreType.DMA,
            pltpu.SemaphoreType.DMA,
            pltpu.SemaphoreType.DMA,
            pltpu.SemaphoreType.DMA,
            pltpu.SemaphoreType.DMA,
            pltpu.SemaphoreType.DMA,
            pltpu.VMEM_SHARED((_NV, _D), jnp.float32),
        ],
    )
    def k(src_hbm, dst_hbm, tu_hbm, tv_hbm, outv_hbm, outu_hbm,
          gidx, sidx, r0, r1, r2, r3, r4,
          g0, g1, g2, g3, g4, t0, t1, t2, t3, t4, acc):
        rows = (r0, r1, r2, r3, r4)
        gsem = (g0, g1, g2, g3, g4)
        tsem = (t0, t1, t2, t3, t4)
        c = lax.axis_index("c")
        s = lax.axis_index("s")

        def _zrow(r, carry):
            for j in range(_D // 16):
                r0[r, pl.ds(j * 16, 16)] = jnp.zeros((16,), jnp.float32)
            return carry
        lax.fori_loop(0, _CH, _zrow, 0)

        def run_dir(g4d_hbm, s4d_hbm, table_hbm, out_hbm,
                    per_tile, plan, extra_plan):
            base = s * per_tile
            off = 0
            for ch in plan:
                pltpu.sync_copy(r0.at[pl.ds(0, ch)],
                                acc.at[pl.ds(base + off, ch)])
                off += ch

            @pl.when(s == 15)
            def _():
                o = per_tile
                for ch in extra_plan:
                    pltpu.sync_copy(r0.at[pl.ds(0, ch)],
                                    acc.at[pl.ds(15 * per_tile + o, ch)])
                    o += ch
            plsc.subcore_barrier()

            g4d_t = g4d_hbm.at[s]
            s4d_t = s4d_hbm.at[s]

            def block(blk, carry):
                pltpu.sync_copy(g4d_t.at[blk], gidx)
                pltpu.sync_copy(s4d_t.at[blk], sidx)
                for b in range(_NBUF):
                    pltpu.async_copy(table_hbm.at[gidx.at[b]],
                                     rows[b], gsem[b])

                def inner(io, cy):
                    ibase = io * _NBUF
                    for b in range(_NBUF):
                        i = ibase + b
                        bp = (b - 1) % _NBUF
                        pltpu.make_async_copy(
                            table_hbm.at[pl.ds(0, _CH)],
                            rows[b], gsem[b]).wait()
                        pltpu.async_copy(rows[b], acc.at[sidx.at[i]],
                                         tsem[b], add=True)
                        jg = i - 1 + _NBUF

                        @pl.when((i >= 1) & (jg < _BLK))
                        def _():
                            pltpu.make_async_copy(
                                table_hbm.at[pl.ds(0, _CH)],
                                rows[bp], tsem[bp]).wait()
                            pltpu.async_copy(table_hbm.at[gidx.at[jg]],
                                             rows[bp], gsem[bp])
                    return cy
                lax.fori_loop(0, _BLK // _NBUF, inner, 0)
                for b in range(_NBUF):
                    pltpu.make_async_copy(
                        table_hbm.at[pl.ds(0, _CH)],
                        rows[b], tsem[b]).wait()
                return carry
            lax.fori_loop(0, _NBLK, block, 0)
            plsc.subcore_barrier()

            def copy_out(row0, ch):
                pltpu.sync_copy(acc.at[pl.ds(row0, ch)],
                                r0.at[pl.ds(0, ch)])
                pltpu.sync_copy(r0.at[pl.ds(0, ch)],
                                out_hbm.at[pl.ds(row0, ch)])

            off = 0
            for ch in plan:
                copy_out(base + off, ch)
                off += ch

            @pl.when(s == 15)
            def _():
                o = per_tile
                for ch in extra_plan:
                    copy_out(15 * per_tile + o, ch)
                    o += ch

        @pl.when(c == 0)
        def _():
            run_dir(dst_hbm, src_hbm, tu_hbm, outv_hbm,
                    624, (40,) * 15 + (24,), (16,))

        @pl.when(c == 1)
        def _():
            run_dir(src_hbm, dst_hbm, tv_hbm, outu_hbm,
                    120, (40, 40, 40), (40, 40))

    return k(src4d, dst4d, table_u, table_v)


def _sc_edge_gather(lflat, src3d2, dst3d2):
    mesh = plsc.VectorSubcoreMesh(core_axis_name="c", subcore_axis_name="s")

    @functools.partial(
        pl.kernel, mesh=mesh,
        out_type=jax.ShapeDtypeStruct((32, _CPT2, _CH2), jnp.float32),
        scratch_types=[
            pltpu.VMEM((_CPT2, _CH2), jnp.int32),
            pltpu.VMEM((_CPT2, _CH2), jnp.int32),
            pltpu.VMEM((_CPT2, _CH2), jnp.int32),
            pltpu.VMEM((_CPT2, _CH2), jnp.float32),
            pltpu.SemaphoreType.DMA,
            pltpu.SemaphoreType.DMA,
            pltpu.SemaphoreType.DMA,
            pltpu.SemaphoreType.DMA,
            pltpu.SemaphoreType.DMA,
        ],
    )
    def k(lflat_hbm, src_hbm, dst_hbm, out_hbm,
          sbuf, dbuf, kbuf, lbuf, g0, g1, g2, g3, g4):
        gsem = (g0, g1, g2, g3, g4)
        c = lax.axis_index("c")
        s = lax.axis_index("s")
        w = c * _TILES + s
        pltpu.sync_copy(src_hbm.at[w], sbuf)
        pltpu.sync_copy(dst_hbm.at[w], dbuf)

        def krow(r, carry):
            for j in range(_CH2 // 16):
                sl = pl.ds(j * 16, 16)
                kbuf[r, sl] = sbuf[r, sl] * _NUP + dbuf[r, sl]
            return carry
        lax.fori_loop(0, _CPT2, krow, 0)

        for b in range(_NBUF2):
            pltpu.async_copy(lflat_hbm.at[kbuf.at[b]], lbuf.at[b], gsem[b])

        def outer(io, carry):
            ibase = io * _NBUF2
            for b in range(_NBUF2):
                i = ibase + b
                pltpu.make_async_copy(
                    lflat_hbm.at[pl.ds(0, _CH2)], lbuf.at[i], gsem[b]).wait()
                nxt = i + _NBUF2

                @pl.when(nxt < _CPT2)
                def _():
                    pltpu.async_copy(
                        lflat_hbm.at[kbuf.at[nxt]], lbuf.at[nxt], gsem[b])
            return carry
        lax.fori_loop(0, _CPT2 // _NBUF2, outer, 0)
        pltpu.sync_copy(lbuf, out_hbm.at[w])

    return k(lflat, src3d2, dst3d2)


def kernel(src_v, dst_u, x_v, x_u, shuff_x_v, shuff_x_u,
           W1v, W1u, b1v, b1u, alpha1, W2v, W2u, b2v, b2u, Wb):
    src4d = src_v.reshape(_TILES, _NBLK, _BLK, _CH)
    dst4d = dst_u.reshape(_TILES, _NBLK, _BLK, _CH)
    src3d2 = src_v.reshape(32, _CPT2, _CH2)
    dst3d2 = dst_u.reshape(32, _CPT2, _CH2)
    b1v_r = b1v.reshape(1, _D)
    b1u_r = b1u.reshape(1, _D)
    b2v_r = b2v.reshape(1, _D)
    b2u_r = b2u.reshape(1, _D)
    al_r = jnp.broadcast_to(alpha1, (1, _D)).astype(jnp.float32)
    wv_cat = jnp.concatenate([W1v, W2v[_D:, :]], axis=1)
    wu_cat = jnp.concatenate([W1u, W2u[_D:, :]], axis=1)

    hv_cat, hu_cat = _tc_mm_pair(x_v, x_u, wv_cat, wu_cat)
    hv, gxv = hv_cat[:, :_D], hv_cat[:, _D:]
    hu, gxu = hu_cat[:, :_D], hu_cat[:, _D:]

    sv1, su1 = _sc_segsum_pair(src4d, dst4d, hu, hv)

    gv, gu = _tc_layer2(sv1, su1, gxv, gxu, W2v[:_D, :], W2u[:_D, :],
                        b1v_r, b1u_r, al_r)

    sv2, su2 = _sc_segsum_pair(src4d, dst4d, gu, gv)

    su2p = jnp.pad(su2, ((0, _NUP - _NU), (0, 0)))
    ve2, ue2, big_l, s0 = _tc_disc_logits(sv2, su2p, b2v_r, b2u_r, Wb)

    l2d = _sc_edge_gather(big_l.reshape(-1), src3d2, dst3d2)
    sneg, spos = _tc_edge_reduce(l2d.reshape(_E // _D, _D))

    n = float(_NV) * float(_NU)
    tsum = float(_E)
    pos_weight = (n - tsum) / tsum
    norm = n / (n - tsum)
    s0_real = s0[0, 0] + float(_NV * (_NUP - _NU)) * 0.6931471805599453
    sum_bce = -s0_real + sneg[0, 0] - pos_weight * spos[0, 0]
    loss = norm * sum_bce / n
    return ve2, ue2, loss


# speedup vs baseline: 11.0455x; 1.0838x over previous
import functools

import jax
import jax.numpy as jnp
from jax import lax
from jax.experimental import pallas as pl
from jax.experimental.pallas import tpu as pltpu
from jax.experimental.pallas import tpu_sc as plsc

_NV, _NU, _E = 10000, 2000, 320000
_D = 128
_CH = 40
_NBLK = 10
_BLK = 50
_NBUF = 5
_TILES = 16
_CH2 = 80
_CPT2 = _E // _CH2 // 32
_NBUF2 = 5


def _mm_pair_body(xv_ref, xu_ref, wv_ref, wu_ref, ov_ref, ou_ref):
    ov_ref[...] = jnp.dot(xv_ref[...], wv_ref[...],
                          preferred_element_type=jnp.float32)
    ou_ref[...] = jnp.dot(xu_ref[...], wu_ref[...],
                          preferred_element_type=jnp.float32)


def _tc_mm_pair(x_v, x_u, wv, wu):
    return pl.pallas_call(
        _mm_pair_body,
        out_shape=(jax.ShapeDtypeStruct((_NV, 2 * _D), jnp.float32),
                   jax.ShapeDtypeStruct((_NU, 2 * _D), jnp.float32)),
    )(x_v, x_u, wv, wu)


def _layer2_body(sv_ref, su_ref, gxv_ref, gxu_ref, wv_ref, wu_ref,
                 bv_ref, bu_ref, al_ref, gv_ref, gu_ref):
    al = al_ref[...]
    av = sv_ref[0:_NV, :] + bv_ref[...]
    av = jnp.where(av >= 0.0, av, al * av)
    gv_ref[...] = jnp.dot(av, wv_ref[...],
                          preferred_element_type=jnp.float32) + gxv_ref[...]
    au = su_ref[0:_NU, :] + bu_ref[...]
    au = jnp.where(au >= 0.0, au, al * au)
    gu_ref[...] = jnp.dot(au, wu_ref[...],
                          preferred_element_type=jnp.float32) + gxu_ref[...]


def _tc_layer2(sv1, su1, gxv, gxu, w2v_top, w2u_top, b1v, b1u, alpha):
    return pl.pallas_call(
        _layer2_body,
        out_shape=(jax.ShapeDtypeStruct((_NV, _D), jnp.float32),
                   jax.ShapeDtypeStruct((_NU, _D), jnp.float32)),
    )(sv1, su1, gxv, gxu, w2v_top, w2u_top, b1v, b1u, alpha)


def _ls(x):
    return jnp.minimum(x, 0.0) - jnp.log(1.0 + jnp.exp(-jnp.abs(x)))


_BM = 400
_NUP = 2048


def _disc_logits_body(sv_ref, su_ref, bv_ref, bu_ref, wb_ref,
                      ve2_ref, ue2_ref, l_ref, s0_ref):
    pid = pl.program_id(0)
    v = sv_ref[...] + bv_ref[...]
    ve2_ref[...] = v
    rid = lax.broadcasted_iota(jnp.int32, (_NUP, _D), 0)
    u = jnp.where(rid < _NU, su_ref[...] + bu_ref[...], 0.0)

    @pl.when(pid == 0)
    def _():
        ue2_ref[...] = u[0:_NU, :]
    av = jnp.dot(v, wb_ref[...], preferred_element_type=jnp.float32)
    band = lax.dot_general(av, u, (((1,), (1,)), ((), ())),
                           preferred_element_type=jnp.float32)
    l_ref[...] = band.reshape(_BM * _NUP // 128, 128)
    part = jnp.sum(_ls(-band))
    s0_ref[...] = jnp.where(pid == 0, part, s0_ref[...] + part)


def _tc_disc_logits(sv2, su2p, b2v, b2u, wb):
    nb = _NV // _BM
    return pl.pallas_call(
        _disc_logits_body,
        grid=(nb,),
        in_specs=[pl.BlockSpec((_BM, _D), lambda i: (i, 0)),
                  pl.BlockSpec((_NUP, _D), lambda i: (0, 0)),
                  pl.BlockSpec((1, _D), lambda i: (0, 0)),
                  pl.BlockSpec((1, _D), lambda i: (0, 0)),
                  pl.BlockSpec((_D, _D), lambda i: (0, 0))],
        out_specs=(pl.BlockSpec((_BM, _D), lambda i: (i, 0)),
                   pl.BlockSpec((_NU, _D), lambda i: (0, 0)),
                   pl.BlockSpec((_BM * _NUP // 128, 128), lambda i: (i, 0)),
                   pl.BlockSpec((1, 1), lambda i: (0, 0))),
        out_shape=(jax.ShapeDtypeStruct((_NV, _D), jnp.float32),
                   jax.ShapeDtypeStruct((_NU, _D), jnp.float32),
                   jax.ShapeDtypeStruct((_NV * _NUP // 128, 128), jnp.float32),
                   jax.ShapeDtypeStruct((1, 1), jnp.float32)),
    )(sv2, su2p, b2v, b2u, wb)


def _edge_reduce_body(l_ref, sneg_ref, spos_ref):
    x = l_ref[...]
    sneg_ref[...] = jnp.sum(_ls(-x)).reshape(1, 1)
    spos_ref[...] = jnp.sum(_ls(x)).reshape(1, 1)


def _tc_edge_reduce(l2d):
    return pl.pallas_call(
        _edge_reduce_body,
        out_shape=(jax.ShapeDtypeStruct((1, 1), jnp.float32),
                   jax.ShapeDtypeStruct((1, 1), jnp.float32)),
    )(l2d)


def _sc_segsum_pair(src4d, dst4d, table_u, table_v):
    mesh = plsc.VectorSubcoreMesh(core_axis_name="c", subcore_axis_name="s")

    @functools.partial(
        pl.kernel, mesh=mesh,
        out_type=(jax.ShapeDtypeStruct((_NV, _D), jnp.float32),
                  jax.ShapeDtypeStruct((_NU, _D), jnp.float32)),
        scratch_types=[
            pltpu.VMEM((_BLK, _CH), jnp.int32),
            pltpu.VMEM((_BLK, _CH), jnp.int32),
            pltpu.VMEM((_CH, _D), jnp.float32),
            pltpu.VMEM((_CH, _D), jnp.float32),
            pltpu.VMEM((_CH, _D), jnp.float32),
            pltpu.VMEM((_CH, _D), jnp.float32),
            pltpu.VMEM((_CH, _D), jnp.float32),
            pltpu.SemaphoreType.DMA,
            pltpu.SemaphoreType.DMA,
            pltpu.SemaphoreType.DMA,
            pltpu.SemaphoreType.DMA,
            pltpu.SemaphoreType.DMA,
            pltpu.SemaphoreType.DMA,
            pltpu.SemaphoreType.DMA,
            pltpu.SemaphoreType.DMA,
            pltpu.SemaphoreType.DMA,
            pltpu.SemaphoreType.DMA,
            pltpu.VMEM_SHARED((_NV, _D), jnp.float32),
        ],
    )
    def k(src_hbm, dst_hbm, tu_hbm, tv_hbm, outv_hbm, outu_hbm,
          gidx, sidx, r0, r1, r2, r3, r4,
          g0, g1, g2, g3, g4, t0, t1, t2, t3, t4, acc):
        rows = (r0, r1, r2, r3, r4)
        gsem = (g0, g1, g2, g3, g4)
        tsem = (t0, t1, t2, t3, t4)
        c = lax.axis_index("c")
        s = lax.axis_index("s")

        def _zrow(r, carry):
            for j in range(_D // 16):
                r0[r, pl.ds(j * 16, 16)] = jnp.zeros((16,), jnp.float32)
            return carry
        lax.fori_loop(0, _CH, _zrow, 0)

        def run_dir(g4d_hbm, s4d_hbm, table_hbm, out_hbm,
                    per_tile, plan, extra_plan):
            base = s * per_tile
            off = 0
            for ch in plan:
                pltpu.sync_copy(r0.at[pl.ds(0, ch)],
                                acc.at[pl.ds(base + off, ch)])
                off += ch

            @pl.when(s == 15)
            def _():
                o = per_tile
                for ch in extra_plan:
                    pltpu.sync_copy(r0.at[pl.ds(0, ch)],
                                    acc.at[pl.ds(15 * per_tile + o, ch)])
                    o += ch
            plsc.subcore_barrier()

            g4d_t = g4d_hbm.at[s]
            s4d_t = s4d_hbm.at[s]

            def block(blk, carry):
                pltpu.sync_copy(g4d_t.at[blk], gidx)
                pltpu.sync_copy(s4d_t.at[blk], sidx)
                for b in range(_NBUF):
                    pltpu.async_copy(table_hbm.at[gidx.at[b]],
                                     rows[b], gsem[b])

                def inner(io, cy):
                    ibase = io * _NBUF
                    for b in range(_NBUF):
                        i = ibase + b
                        bp = (b - 1) % _NBUF
                        pltpu.make_async_copy(
                            table_hbm.at[pl.ds(0, _CH)],
                            rows[b], gsem[b]).wait()
                        pltpu.async_copy(rows[b], acc.at[sidx.at[i]],
                                         tsem[b], add=True)
                        jg = i - 1 + _NBUF

                        @pl.when((i >= 1) & (jg < _BLK))
                        def _():
                            pltpu.make_async_copy(
                                table_hbm.at[pl.ds(0, _CH)],
                                rows[bp], tsem[bp]).wait()
                            pltpu.async_copy(table_hbm.at[gidx.at[jg]],
                                             rows[bp], gsem[bp])
                    return cy
                lax.fori_loop(0, _BLK // _NBUF, inner, 0)
                for b in range(_NBUF):
                    pltpu.make_async_copy(
                        table_hbm.at[pl.ds(0, _CH)],
                        rows[b], tsem[b]).wait()
                return carry
            lax.fori_loop(0, _NBLK, block, 0)
            plsc.subcore_barrier()

            def copy_out(row0, ch):
                pltpu.sync_copy(acc.at[pl.ds(row0, ch)],
                                r0.at[pl.ds(0, ch)])
                pltpu.sync_copy(r0.at[pl.ds(0, ch)],
                                out_hbm.at[pl.ds(row0, ch)])

            off = 0
            for ch in plan:
                copy_out(base + off, ch)
                off += ch

            @pl.when(s == 15)
            def _():
                o = per_tile
                for ch in extra_plan:
                    copy_out(15 * per_tile + o, ch)
                    o += ch

        @pl.when(c == 0)
        def _():
            run_dir(dst_hbm, src_hbm, tu_hbm, outv_hbm,
                    624, (40,) * 15 + (24,), (16,))

        @pl.when(c == 1)
        def _():
            run_dir(src_hbm, dst_hbm, tv_hbm, outu_hbm,
                    120, (40, 40, 40), (40, 40))

    return k(src4d, dst4d, table_u, table_v)


def _sc_edge_gather(lflat, src3d2, dst3d2):
    mesh = plsc.VectorSubcoreMesh(core_axis_name="c", subcore_axis_name="s")

    @functools.partial(
        pl.kernel, mesh=mesh,
        out_type=jax.ShapeDtypeStruct((32, _CPT2, _CH2), jnp.float32),
        scratch_types=[
            pltpu.VMEM((_CPT2, _CH2), jnp.int32),
            pltpu.VMEM((_CPT2, _CH2), jnp.int32),
            pltpu.VMEM((_CPT2, _CH2), jnp.int32),
            pltpu.VMEM((_CPT2, _CH2), jnp.float32),
            pltpu.SemaphoreType.DMA,
            pltpu.SemaphoreType.DMA,
            pltpu.SemaphoreType.DMA,
            pltpu.SemaphoreType.DMA,
            pltpu.SemaphoreType.DMA,
        ],
    )
    def k(lflat_hbm, src_hbm, dst_hbm, out_hbm,
          sbuf, dbuf, kbuf, lbuf, g0, g1, g2, g3, g4):
        gsem = (g0, g1, g2, g3, g4)
        c = lax.axis_index("c")
        s = lax.axis_index("s")
        w = c * _TILES + s
        pltpu.sync_copy(src_hbm.at[w], sbuf)
        pltpu.sync_copy(dst_hbm.at[w], dbuf)

        def krow(r, carry):
            for j in range(_CH2 // 16):
                sl = pl.ds(j * 16, 16)
                kbuf[r, sl] = sbuf[r, sl] * _NUP + dbuf[r, sl]
            return carry
        lax.fori_loop(0, _CPT2, krow, 0)

        for b in range(_NBUF2):
            pltpu.async_copy(lflat_hbm.at[kbuf.at[b]], lbuf.at[b], gsem[b])

        def outer(io, carry):
            ibase = io * _NBUF2
            for b in range(_NBUF2):
                i = ibase + b
                pltpu.make_async_copy(
                    lflat_hbm.at[pl.ds(0, _CH2)], lbuf.at[i], gsem[b]).wait()
                nxt = i + _NBUF2

                @pl.when(nxt < _CPT2)
                def _():
                    pltpu.async_copy(
                        lflat_hbm.at[kbuf.at[nxt]], lbuf.at[nxt], gsem[b])
            return carry
        lax.fori_loop(0, _CPT2 // _NBUF2, outer, 0)
        pltpu.sync_copy(lbuf, out_hbm.at[w])

    return k(lflat, src3d2, dst3d2)


def kernel(src_v, dst_u, x_v, x_u, shuff_x_v, shuff_x_u,
           W1v, W1u, b1v, b1u, alpha1, W2v, W2u, b2v, b2u, Wb):
    src4d = src_v.reshape(_TILES, _NBLK, _BLK, _CH)
    dst4d = dst_u.reshape(_TILES, _NBLK, _BLK, _CH)
    src3d2 = src_v.reshape(32, _CPT2, _CH2)
    dst3d2 = dst_u.reshape(32, _CPT2, _CH2)
    b1v_r = b1v.reshape(1, _D)
    b1u_r = b1u.reshape(1, _D)
    b2v_r = b2v.reshape(1, _D)
    b2u_r = b2u.reshape(1, _D)
    al_r = jnp.broadcast_to(alpha1, (1, _D)).astype(jnp.float32)
    wv_cat = jnp.concatenate([W1v, W2v[_D:, :]], axis=1)
    wu_cat = jnp.concatenate([W1u, W2u[_D:, :]], axis=1)

    hv_cat, hu_cat = _tc_mm_pair(x_v, x_u, wv_cat, wu_cat)
    hv, gxv = hv_cat[:, :_D], hv_cat[:, _D:]
    hu, gxu = hu_cat[:, :_D], hu_cat[:, _D:]

    sv1, su1 = _sc_segsum_pair(src4d, dst4d, hu, hv)

    gv, gu = _tc_layer2(sv1, su1, gxv, gxu, W2v[:_D, :], W2u[:_D, :],
                        b1v_r, b1u_r, al_r)

    sv2, su2 = _sc_segsum_pair(src4d, dst4d, gu, gv)

    su2p = jnp.pad(su2, ((0, _NUP - _NU), (0, 0)))
    ve2, ue2, big_l, s0 = _tc_disc_logits(sv2, su2p, b2v_r, b2u_r, Wb)

    l2d = _sc_edge_gather(big_l.reshape(-1), src3d2, dst3d2)
    sneg, spos = _tc_edge_reduce(l2d.reshape(_E // _D, _D))

    n = float(_NV) * float(_NU)
    tsum = float(_E)
    pos_weight = (n - tsum) / tsum
    norm = n / (n - tsum)
    s0_real = s0[0, 0] + float(_NV * (_NUP - _NU)) * 0.6931471805599453
    sum_bce = -s0_real + sneg[0, 0] - pos_weight * spos[0, 0]
    loss = norm * sum_bce / n
    return ve2, ue2, loss
